# Initial kernel scaffold; baseline (speedup 1.0000x reference)
import jax
import jax.numpy as jnp
from jax.experimental import pallas as pl


def kernel(tokens, channel_embed, timestamps, patch_size, input_res):
    raise NotImplementedError("write your pallas kernel here")


# speedup vs baseline: 2.9538x; 2.9538x over previous
import functools

import numpy as np
import jax
import jax.numpy as jnp
from jax import lax
from jax.experimental import pallas as pl
from jax.experimental.pallas import tpu as pltpu
from jax.experimental.pallas import tpu_sc as plsc

EMBED = 768
MAX_SEQ = 24
BASE_GSD = 10
_N = EMBED // 4


def _np_sincos_1d(dim, pos):
    omega = np.arange(dim // 2, dtype=np.float32) / (dim / 2.0)
    omega = (1.0 / (10000.0 ** omega)).astype(np.float32)
    out = pos[:, None].astype(np.float32) * omega
    return np.concatenate([np.sin(out), np.cos(out)], axis=-1).astype(np.float32)


def _np_month_table(dim):
    angles = np.arange(0, 13, dtype=np.float32) / (12.0 / (2.0 * np.pi))
    sin_t = np.stack([np.sin(angles)] * (dim // 2), axis=-1)
    cos_t = np.stack([np.cos(angles)] * (dim // 2), axis=-1)
    return np.concatenate([sin_t[:-1], cos_t[:-1]], axis=-1).astype(np.float32)


_PE = _np_sincos_1d(_N, np.arange(MAX_SEQ))
_MT = _np_month_table(_N)


def _sc_row_gather(table, idx, per_w):
    rows_total = idx.shape[0]
    width = table.shape[1]
    n_workers = rows_total // per_w
    mesh = plsc.VectorSubcoreMesh(core_axis_name="c", subcore_axis_name="s")

    @functools.partial(
        pl.kernel,
        mesh=mesh,
        out_type=jax.ShapeDtypeStruct((rows_total, width), jnp.float32),
        scratch_types=[
            pltpu.VMEM((per_w,), jnp.int32),
            pltpu.VMEM((per_w, width), jnp.float32),
            pltpu.SemaphoreType.DMA,
        ],
    )
    def k(table_hbm, idx_hbm, out_hbm, idx_v, rows_v, sem):
        wid = lax.axis_index("s") * 2 + lax.axis_index("c")

        @pl.when(wid < n_workers)
        def _():
            base = wid * per_w
            pltpu.sync_copy(idx_hbm.at[pl.ds(base, per_w)], idx_v)
            pltpu.async_copy(table_hbm.at[idx_v], rows_v, sem).wait()
            pltpu.sync_copy(rows_v, out_hbm.at[pl.ds(base, per_w)])

    return k(table, idx)


def _tc_stream_add(tok3, add3, sef):
    def body(t_ref, a_ref, s_ref, o_ref):
        o_ref[...] = t_ref[...] + a_ref[...] + s_ref[...][:, :, None, :]

    return pl.pallas_call(
        body,
        grid=(tok3.shape[0],),
        in_specs=[
            pl.BlockSpec((1, 12, 3, EMBED), lambda i: (i, 0, 0, 0)),
            pl.BlockSpec((1, 12, 3, EMBED), lambda i: (i // 64, 0, 0, 0)),
            pl.BlockSpec((1, 1, EMBED), lambda i: (i % 64, 0, 0)),
        ],
        out_specs=pl.BlockSpec((1, 12, 3, EMBED), lambda i: (i, 0, 0, 0)),
        out_shape=jax.ShapeDtypeStruct(tok3.shape, jnp.float32),
    )(tok3, add3, sef)


def kernel(tokens, channel_embed, timestamps, patch_size, input_res):
    b, h, w, t, bs, d = tokens.shape
    n = d // 4

    n_months = _MT.shape[0]
    ce = jnp.broadcast_to(channel_embed[None, None, :, :],
                          (n_months, t, bs, n))
    pe = jnp.broadcast_to(jnp.asarray(_PE[:t])[None, :, None, :],
                          (n_months, t, bs, n))
    mt = jnp.broadcast_to(jnp.asarray(_MT)[:, None, None, :],
                          (n_months, t, bs, n))
    zeros = jnp.zeros((n_months, t, bs, n), jnp.float32)
    addfull = jnp.concatenate([ce, pe, mt, zeros], axis=-1)
    addfull = addfull.reshape(n_months * t * bs, d)

    months = timestamps[:, :, 1].astype(jnp.int32)
    idx = (months[:, :, None] * (t * bs)
           + jnp.arange(t, dtype=jnp.int32)[None, :, None] * bs
           + jnp.arange(bs, dtype=jnp.int32)[None, None, :])
    add3 = _sc_row_gather(addfull, idx.reshape(-1), per_w=16)
    add3 = add3.reshape(b, t, bs, d)

    ratio = (jnp.asarray(input_res, jnp.float32)
             * jnp.asarray(patch_size, jnp.float32)) / BASE_GSD
    gh = jnp.repeat(jnp.arange(h, dtype=jnp.float32), w)
    gw = jnp.tile(jnp.arange(w, dtype=jnp.float32), h)
    omega = jnp.asarray(
        (1.0 / (10000.0 ** (np.arange(n // 4, dtype=np.float32)
                            / (n / 4.0)))).astype(np.float32))
    ph = gh[:, None] * ratio * omega
    pw = gw[:, None] * ratio * omega
    se = jnp.concatenate(
        [jnp.sin(ph), jnp.cos(ph), jnp.sin(pw), jnp.cos(pw)], axis=-1)
    sef = jnp.concatenate(
        [jnp.zeros((h * w, 3 * n), jnp.float32), se], axis=-1)
    sef = sef.reshape(h * w, 1, d)

    tok3 = tokens.reshape(b * h * w, t, bs, d)
    out = _tc_stream_add(tok3, add3, sef)
    return out.reshape(tokens.shape)


# speedup vs baseline: 5.6669x; 1.9185x over previous
import functools

import numpy as np
import jax
import jax.numpy as jnp
from jax import lax
from jax.experimental import pallas as pl
from jax.experimental.pallas import tpu as pltpu
from jax.experimental.pallas import tpu_sc as plsc

EMBED = 768
MAX_SEQ = 24
BASE_GSD = 10
_N = EMBED // 4


def _np_sincos_1d(dim, pos):
    omega = np.arange(dim // 2, dtype=np.float32) / (dim / 2.0)
    omega = (1.0 / (10000.0 ** omega)).astype(np.float32)
    out = pos[:, None].astype(np.float32) * omega
    return np.concatenate([np.sin(out), np.cos(out)], axis=-1).astype(np.float32)


def _np_month_table(dim):
    angles = np.arange(0, 13, dtype=np.float32) / (12.0 / (2.0 * np.pi))
    sin_t = np.stack([np.sin(angles)] * (dim // 2), axis=-1)
    cos_t = np.stack([np.cos(angles)] * (dim // 2), axis=-1)
    return np.concatenate([sin_t[:-1], cos_t[:-1]], axis=-1).astype(np.float32)


_PE = _np_sincos_1d(_N, np.arange(MAX_SEQ))
_MT = _np_month_table(_N)


def _sc_row_gather(table, idx, per_w):
    rows_total = idx.shape[0]
    width = table.shape[1]
    n_workers = rows_total // per_w
    mesh = plsc.VectorSubcoreMesh(core_axis_name="c", subcore_axis_name="s")

    @functools.partial(
        pl.kernel,
        mesh=mesh,
        out_type=jax.ShapeDtypeStruct((rows_total, width), jnp.float32),
        scratch_types=[
            pltpu.VMEM((per_w,), jnp.int32),
            pltpu.VMEM((per_w, width), jnp.float32),
            pltpu.SemaphoreType.DMA,
        ],
    )
    def k(table_hbm, idx_hbm, out_hbm, idx_v, rows_v, sem):
        wid = lax.axis_index("s") * 2 + lax.axis_index("c")

        @pl.when(wid < n_workers)
        def _():
            base = wid * per_w
            pltpu.sync_copy(idx_hbm.at[pl.ds(base, per_w)], idx_v)
            pltpu.async_copy(table_hbm.at[idx_v], rows_v, sem).wait()
            pltpu.sync_copy(rows_v, out_hbm.at[pl.ds(base, per_w)])

    return k(table, idx)


def _tc_stream_add(tokens, add3, sef):
    b, h, w, t, bs, d = tokens.shape

    def body(t_ref, a_ref, s_ref, o_ref):
        a = a_ref[...][None, None]
        s = s_ref[...][None, None, :, :, None, :]
        o_ref[...] = t_ref[...] + a + s

    return pl.pallas_call(
        body,
        grid=(b * h,),
        in_specs=[
            pl.BlockSpec((1, 1, w, t, bs, d), lambda i: (i // h, i % h, 0, 0, 0, 0)),
            pl.BlockSpec((1, t, bs, d), lambda i: (i // h, 0, 0, 0)),
            pl.BlockSpec((w, 1, d), lambda i: (i % h, 0, 0)),
        ],
        out_specs=pl.BlockSpec((1, 1, w, t, bs, d), lambda i: (i // h, i % h, 0, 0, 0, 0)),
        out_shape=jax.ShapeDtypeStruct(tokens.shape, jnp.float32),
    )(tokens, add3, sef)


def kernel(tokens, channel_embed, timestamps, patch_size, input_res):
    b, h, w, t, bs, d = tokens.shape
    n = d // 4

    n_months = _MT.shape[0]
    ce = jnp.broadcast_to(channel_embed[None, None, :, :],
                          (n_months, t, bs, n))
    pe = jnp.broadcast_to(jnp.asarray(_PE[:t])[None, :, None, :],
                          (n_months, t, bs, n))
    mt = jnp.broadcast_to(jnp.asarray(_MT)[:, None, None, :],
                          (n_months, t, bs, n))
    zeros = jnp.zeros((n_months, t, bs, n), jnp.float32)
    addfull = jnp.concatenate([ce, pe, mt, zeros], axis=-1)
    addfull = addfull.reshape(n_months * t * bs, d)

    months = timestamps[:, :, 1].astype(jnp.int32)
    idx = (months[:, :, None] * (t * bs)
           + jnp.arange(t, dtype=jnp.int32)[None, :, None] * bs
           + jnp.arange(bs, dtype=jnp.int32)[None, None, :])
    add3 = _sc_row_gather(addfull, idx.reshape(-1), per_w=16)
    add3 = add3.reshape(b, t, bs, d)

    ratio = (jnp.asarray(input_res, jnp.float32)
             * jnp.asarray(patch_size, jnp.float32)) / BASE_GSD
    gh = jnp.repeat(jnp.arange(h, dtype=jnp.float32), w)
    gw = jnp.tile(jnp.arange(w, dtype=jnp.float32), h)
    omega = jnp.asarray(
        (1.0 / (10000.0 ** (np.arange(n // 4, dtype=np.float32)
                            / (n / 4.0)))).astype(np.float32))
    ph = gh[:, None] * ratio * omega
    pw = gw[:, None] * ratio * omega
    se = jnp.concatenate(
        [jnp.sin(ph), jnp.cos(ph), jnp.sin(pw), jnp.cos(pw)], axis=-1)
    sef = jnp.concatenate(
        [jnp.zeros((h * w, 3 * n), jnp.float32), se], axis=-1)
    sef = sef.reshape(h * w, 1, d)

    return _tc_stream_add(tokens, add3, sef)
